# baseline (speedup 1.0000x reference)
import functools

import jax
import jax.numpy as jnp
from jax import lax
from jax.experimental import pallas as pl
from jax.experimental.pallas import tpu as pltpu
from jax.experimental.pallas import tpu_sc as plsc

NC = 2
NS = 16
L = 16
NW = NC * NS

B = 16384
D = 128
CHUNK = 128
PER_W = B // NW
NCHUNK = PER_W // CHUNK
NBUF = 3


def _gmf_body(users_hbm, items_hbm, utab_hbm, itab_hbm, out_hbm,
              idx_u, idx_i, ru0, ri0, ru1, ri1, ru2, ri2,
              sem_x, sem_g0, sem_g1, sem_g2, sem_o0, sem_o1, sem_o2):
    wid = lax.axis_index("s") * NC + lax.axis_index("c")
    base_w = wid * PER_W
    cx_u = pltpu.async_copy(users_hbm.at[pl.ds(base_w, PER_W)], idx_u, sem_x)
    cx_i = pltpu.async_copy(items_hbm.at[pl.ds(base_w, PER_W)], idx_i, sem_x)
    cx_u.wait()
    cx_i.wait()

    ru = [ru0, ru1, ru2]
    ri = [ri0, ri1, ri2]
    sem_g = [sem_g0, sem_g1, sem_g2]
    sem_o = [sem_o0, sem_o1, sem_o2]

    def start_gathers(c):
        b = c % NBUF
        s = pl.ds(c * CHUNK, CHUNK)
        cu = pltpu.async_copy(utab_hbm.at[idx_u.at[s]], ru[b], sem_g[b])
        ci = pltpu.async_copy(itab_hbm.at[idx_i.at[s]], ri[b], sem_g[b])
        return cu, ci

    gathers = {c: start_gathers(c) for c in range(min(NBUF, NCHUNK))}
    out_copies = {}
    for c in range(NCHUNK):
        b = c % NBUF
        cu, ci = gathers[c]
        cu.wait()
        ci.wait()

        def mul_row(r, carry):
            for j in range(D // L):
                sl = pl.ds(j * L, L)
                ru[b][r, sl] = ru[b][r, sl] * ri[b][r, sl]
            return carry

        lax.fori_loop(0, CHUNK, mul_row, 0)
        out_copies[c] = pltpu.async_copy(
            ru[b], out_hbm.at[pl.ds(base_w + c * CHUNK, CHUNK)], sem_o[b])
        nxt = c + NBUF
        if nxt < NCHUNK:
            out_copies[c].wait()
            gathers[nxt] = start_gathers(nxt)
    for c in range(max(0, NCHUNK - NBUF), NCHUNK):
        out_copies[c].wait()


_gmf = functools.partial(
    pl.kernel,
    out_type=jax.ShapeDtypeStruct((B, D), jnp.float32),
    mesh=plsc.VectorSubcoreMesh(
        core_axis_name="c", subcore_axis_name="s",
        num_cores=NC, num_subcores=NS),
    scratch_types=[
        pltpu.VMEM((PER_W,), jnp.int32),
        pltpu.VMEM((PER_W,), jnp.int32),
        pltpu.VMEM((CHUNK, D), jnp.float32),
        pltpu.VMEM((CHUNK, D), jnp.float32),
        pltpu.VMEM((CHUNK, D), jnp.float32),
        pltpu.VMEM((CHUNK, D), jnp.float32),
        pltpu.VMEM((CHUNK, D), jnp.float32),
        pltpu.VMEM((CHUNK, D), jnp.float32),
        pltpu.SemaphoreType.DMA,
        pltpu.SemaphoreType.DMA,
        pltpu.SemaphoreType.DMA,
        pltpu.SemaphoreType.DMA,
        pltpu.SemaphoreType.DMA,
        pltpu.SemaphoreType.DMA,
        pltpu.SemaphoreType.DMA,
    ],
)(_gmf_body)


def kernel(users, items, user_table, item_table):
    return _gmf(users.astype(jnp.int32), items.astype(jnp.int32),
                user_table, item_table)


# speedup vs baseline: 1.0342x; 1.0342x over previous
import functools

import jax
import jax.numpy as jnp
from jax import lax
from jax.experimental import pallas as pl
from jax.experimental.pallas import tpu as pltpu
from jax.experimental.pallas import tpu_sc as plsc

NC = 2
NS = 16
L = 16
NW = NC * NS

B = 16384
D = 128
CHUNK = 64
PER_W = B // NW
NCHUNK = PER_W // CHUNK
NBUF = 6


def _gmf_body(users_hbm, items_hbm, utab_hbm, itab_hbm, out_hbm, *scratch):
    idx_u, idx_i = scratch[0], scratch[1]
    ru = list(scratch[2:2 + NBUF])
    ri = list(scratch[2 + NBUF:2 + 2 * NBUF])
    sem_x = scratch[2 + 2 * NBUF]
    sem_g = list(scratch[3 + 2 * NBUF:3 + 3 * NBUF])
    sem_o = list(scratch[3 + 3 * NBUF:3 + 4 * NBUF])

    wid = lax.axis_index("s") * NC + lax.axis_index("c")
    base_w = wid * PER_W
    cx_u = pltpu.async_copy(users_hbm.at[pl.ds(base_w, PER_W)], idx_u, sem_x)
    cx_i = pltpu.async_copy(items_hbm.at[pl.ds(base_w, PER_W)], idx_i, sem_x)
    cx_u.wait()
    cx_i.wait()

    def start_gathers(c):
        b = c % NBUF
        s = pl.ds(c * CHUNK, CHUNK)
        cu = pltpu.async_copy(utab_hbm.at[idx_u.at[s]], ru[b], sem_g[b])
        ci = pltpu.async_copy(itab_hbm.at[idx_i.at[s]], ri[b], sem_g[b])
        return cu, ci

    gathers = {c: start_gathers(c) for c in range(min(NBUF, NCHUNK))}
    out_copies = {}
    for c in range(NCHUNK):
        b = c % NBUF
        cu, ci = gathers[c]
        cu.wait()
        ci.wait()

        def mul_row(r, carry):
            for j in range(D // L):
                sl = pl.ds(j * L, L)
                ru[b][r, sl] = ru[b][r, sl] * ri[b][r, sl]
            return carry

        lax.fori_loop(0, CHUNK, mul_row, 0)
        out_copies[c] = pltpu.async_copy(
            ru[b], out_hbm.at[pl.ds(base_w + c * CHUNK, CHUNK)], sem_o[b])
        nxt = c + NBUF
        if nxt < NCHUNK:
            out_copies[c].wait()
            gathers[nxt] = start_gathers(nxt)
    for c in range(max(0, NCHUNK - NBUF), NCHUNK):
        out_copies[c].wait()


_gmf = functools.partial(
    pl.kernel,
    out_type=jax.ShapeDtypeStruct((B, D), jnp.float32),
    mesh=plsc.VectorSubcoreMesh(
        core_axis_name="c", subcore_axis_name="s",
        num_cores=NC, num_subcores=NS),
    scratch_types=(
        [pltpu.VMEM((PER_W,), jnp.int32)] * 2
        + [pltpu.VMEM((CHUNK, D), jnp.float32)] * (2 * NBUF)
        + [pltpu.SemaphoreType.DMA] * (1 + 2 * NBUF)
    ),
)(_gmf_body)


def kernel(users, items, user_table, item_table):
    return _gmf(users.astype(jnp.int32), items.astype(jnp.int32),
                user_table, item_table)


# speedup vs baseline: 1.0425x; 1.0080x over previous
import functools

import jax
import jax.numpy as jnp
from jax import lax
from jax.experimental import pallas as pl
from jax.experimental.pallas import tpu as pltpu
from jax.experimental.pallas import tpu_sc as plsc

NC = 2
NS = 16
L = 16
NW = NC * NS

B = 16384
D = 128
CHUNK = 64
PER_W = B // NW
NCHUNK = PER_W // CHUNK
NBUF = 7


def _gmf_body(users_hbm, items_hbm, utab_hbm, itab_hbm, out_hbm, *scratch):
    idx_u, idx_i = scratch[0], scratch[1]
    ru = list(scratch[2:2 + NBUF])
    ri = list(scratch[2 + NBUF:2 + 2 * NBUF])
    sem_x = scratch[2 + 2 * NBUF]
    sem_g = list(scratch[3 + 2 * NBUF:3 + 3 * NBUF])
    sem_o = list(scratch[3 + 3 * NBUF:3 + 4 * NBUF])

    wid = lax.axis_index("s") * NC + lax.axis_index("c")
    base_w = wid * PER_W
    cx_u = pltpu.async_copy(users_hbm.at[pl.ds(base_w, PER_W)], idx_u, sem_x)
    cx_i = pltpu.async_copy(items_hbm.at[pl.ds(base_w, PER_W)], idx_i, sem_x)
    cx_u.wait()
    cx_i.wait()

    def start_gathers(c):
        b = c % NBUF
        s = pl.ds(c * CHUNK, CHUNK)
        cu = pltpu.async_copy(utab_hbm.at[idx_u.at[s]], ru[b], sem_g[b])
        ci = pltpu.async_copy(itab_hbm.at[idx_i.at[s]], ri[b], sem_g[b])
        return cu, ci

    gathers = {c: start_gathers(c) for c in range(min(NBUF, NCHUNK))}
    out_copies = {}
    for c in range(NCHUNK):
        b = c % NBUF
        cu, ci = gathers[c]
        cu.wait()
        ci.wait()

        def mul_row(r, carry):
            for j in range(D // L):
                sl = pl.ds(j * L, L)
                ru[b][r, sl] = ru[b][r, sl] * ri[b][r, sl]
            return carry

        lax.fori_loop(0, CHUNK, mul_row, 0)
        out_copies[c] = pltpu.async_copy(
            ru[b], out_hbm.at[pl.ds(base_w + c * CHUNK, CHUNK)], sem_o[b])
        nxt = c + NBUF
        if nxt < NCHUNK:
            out_copies[c].wait()
            gathers[nxt] = start_gathers(nxt)
    for c in range(max(0, NCHUNK - NBUF), NCHUNK):
        out_copies[c].wait()


_gmf = functools.partial(
    pl.kernel,
    out_type=jax.ShapeDtypeStruct((B, D), jnp.float32),
    mesh=plsc.VectorSubcoreMesh(
        core_axis_name="c", subcore_axis_name="s",
        num_cores=NC, num_subcores=NS),
    scratch_types=(
        [pltpu.VMEM((PER_W,), jnp.int32)] * 2
        + [pltpu.VMEM((CHUNK, D), jnp.float32)] * (2 * NBUF)
        + [pltpu.SemaphoreType.DMA] * (1 + 2 * NBUF)
    ),
)(_gmf_body)


def kernel(users, items, user_table, item_table):
    return _gmf(users.astype(jnp.int32), items.astype(jnp.int32),
                user_table, item_table)
